# baseline (speedup 1.0000x reference)
import functools
from math import factorial

import jax
import jax.numpy as jnp
import numpy as np
from jax import lax
from jax.experimental import pallas as pl
from jax.experimental.pallas import tpu as pltpu
from jax.experimental.pallas import tpu_sc as plsc

N_NODES = 10000
N_EDGES = 320000
F = 128
N_BASIS = 8
HIDDEN = 64
AVG_NEIGH = 32.0


def _poly_env_coeffs(n0, n1):
    ncoef = n0 + n1 + 2
    A = np.zeros((ncoef, ncoef))
    b = np.zeros(ncoef)
    row = 0
    for k in range(n0 + 1):
        A[row, k] = float(factorial(k))
        b[row] = 1.0 if k == 0 else 0.0
        row += 1
    for k in range(n1 + 1):
        for j in range(k, ncoef):
            A[row, j] = float(factorial(j)) / float(factorial(j - k))
        b[row] = 0.0
        row += 1
    return np.linalg.solve(A, b)

_ENV_HORNER = tuple(float(c) for c in _poly_env_coeffs(5, 2)[::-1])

_z = np.random.RandomState(0).randn(1000000)
_ACT_NORM = float(1.0 / np.sqrt(np.mean((_z / (1.0 + np.exp(-_z))) ** 2)))

_SQRT3 = float(np.sqrt(3.0))
_SQRT2 = float(np.sqrt(2.0))
_PI = float(np.pi)
_INV_SQRT_NEIGH = float(1.0 / np.sqrt(AVG_NEIGH))

_l = np.arange(F // 2)
_SIG_LO = 32 * (_l // 16) + (_l % 16)
_SIGMA = np.concatenate([_SIG_LO, _SIG_LO + 16])


_BE = 1280


def _coef_body(v_ref, w1_ref, w2_ref, w3_ref, out_ref):
    v = v_ref[...]
    len2 = jnp.sum(v * v, axis=1, keepdims=True)
    length = jnp.sqrt(len2)
    safe = jnp.where(length == 0.0, 1.0, length)
    sh = (_SQRT3 / safe) * v

    n = lax.broadcasted_iota(jnp.int32, (_BE, N_BASIS), 1).astype(jnp.float32) + 1.0
    xnz = jnp.where(length == 0.0, 1.0, length)
    basis = _SQRT2 * jnp.where(
        length == 0.0, n * _PI, jnp.sin(n * (_PI * length)) / xnz
    )

    p = jnp.full_like(length, _ENV_HORNER[0])
    for cfl in _ENV_HORNER[1:]:
        p = p * length + cfl
    cutoff = jnp.where(length < 1.0, p, 0.0)

    radial = basis * cutoff

    def _swish(x):
        return _ACT_NORM * x / (1.0 + jnp.exp(-x))

    h = _swish(jnp.dot(radial, w1_ref[...], preferred_element_type=jnp.float32))
    h = _swish(jnp.dot(h, w2_ref[...], preferred_element_type=jnp.float32))
    mix = jnp.dot(h, w3_ref[...], preferred_element_type=jnp.float32)
    mix = mix * _INV_SQRT_NEIGH

    def _pack(half):
        lo_b = lax.bitcast_convert_type(half[:, : F // 2], jnp.uint32)
        hi_b = lax.bitcast_convert_type(half[:, F // 2 :], jnp.uint32)
        lo_r = (lo_b + jnp.uint32(0x8000)) >> 16
        hi_r = (hi_b + jnp.uint32(0x8000)) & jnp.uint32(0xFFFF0000)
        return lax.bitcast_convert_type(lo_r | hi_r, jnp.int32)

    out_ref[0] = _pack(mix[:, :F])
    for k in range(3):
        out_ref[k + 1] = _pack(mix[:, F:] * sh[:, k : k + 1])


def _coef_pallas(vectors, W1s, W2s, W3s):
    grid = N_EDGES // _BE
    return pl.pallas_call(
        _coef_body,
        grid=(grid,),
        in_specs=[
            pl.BlockSpec((_BE, 3), lambda i: (i, 0)),
            pl.BlockSpec((N_BASIS, HIDDEN), lambda i: (0, 0)),
            pl.BlockSpec((HIDDEN, HIDDEN), lambda i: (0, 0)),
            pl.BlockSpec((HIDDEN, 2 * F), lambda i: (0, 0)),
        ],
        out_specs=pl.BlockSpec((4, _BE, F // 2), lambda i: (0, i, 0)),
        out_shape=jax.ShapeDtypeStruct((4, N_EDGES, F // 2), jnp.int32),
    )(vectors, W1s, W2s, W3s)


_NC = 2
_NS = 16
_C = 80
_EDGES_PER_TILE = N_EDGES // _NS
_NCHUNK = _EDGES_PER_TILE // _C
_NP = 10112
_ROWS_PER_TILE = _NP // _NS
_HA = 48
_HB = _C - _HA


def _sc_body(nf_hbm, send_hbm, recv_hbm, coef_hbm, zeros_hbm, out_hbm,
             s_v, r_v, nf_v, cf_v, msg_v, acc_sh, sem):
    c = lax.axis_index("c")
    s = lax.axis_index("s")
    row0 = s * _ROWS_PER_TILE
    ebase = s * _EDGES_PER_TILE
    mask_hi = jnp.int32(-65536)

    for p in range(2):
        q = 2 * p + c

        pltpu.sync_copy(zeros_hbm, acc_sh.at[pl.ds(row0, _ROWS_PER_TILE)])
        plsc.subcore_barrier()

        def _edge_batch(j, carry):
            e0 = ebase + j * _C
            pltpu.sync_copy(send_hbm.at[pl.ds(e0, _C)], s_v)
            pltpu.sync_copy(recv_hbm.at[pl.ds(e0, _C)], r_v)
            pltpu.sync_copy(
                coef_hbm.at[pl.ds(q * N_EDGES + e0, _C)], cf_v
            )
            pltpu.async_copy(nf_hbm.at[s_v], nf_v, sem).wait()

            def _mul_row(i, c2):
                for g in range(F // 32):
                    w = cf_v[i, pl.ds(g * 16, 16)]
                    cf_lo = lax.bitcast_convert_type(w << 16, jnp.float32)
                    cf_hi = lax.bitcast_convert_type(w & mask_hi, jnp.float32)
                    sl_lo = pl.ds(g * 32, 16)
                    sl_hi = pl.ds(g * 32 + 16, 16)
                    msg_v[i, sl_lo] = nf_v[i, sl_lo] * cf_lo
                    msg_v[i, sl_hi] = nf_v[i, sl_hi] * cf_hi
                return c2

            lax.fori_loop(0, _C, _mul_row, 0, unroll=2)
            pltpu.sync_copy(msg_v, acc_sh.at[r_v], add=True)
            return carry

        lax.fori_loop(0, _NCHUNK, _edge_batch, 0)
        plsc.subcore_barrier()

        pltpu.sync_copy(
            acc_sh.at[pl.ds(row0, _ROWS_PER_TILE)],
            out_hbm.at[pl.ds(q * _NP + row0, _ROWS_PER_TILE)],
        )
        if p == 0:
            plsc.subcore_barrier()


@functools.cache
def _get_sc_kernel():
    return functools.partial(
        pl.kernel,
        mesh=plsc.VectorSubcoreMesh(core_axis_name="c", subcore_axis_name="s"),
        out_type=jax.ShapeDtypeStruct((4 * _NP, F), jnp.float32),
        scratch_types=[
            pltpu.VMEM((_C,), jnp.int32),
            pltpu.VMEM((_C,), jnp.int32),
            pltpu.VMEM((_C, F), jnp.float32),
            pltpu.VMEM((_C, F // 2), jnp.int32),
            pltpu.VMEM((_C, F), jnp.float32),
            pltpu.VMEM_SHARED((_NP, F), jnp.float32),
            pltpu.SemaphoreType.DMA,
        ],
    )(_sc_body)


def kernel(vectors, node_feats, senders, receivers, W1, W2, W3):
    W1s = W1 / jnp.sqrt(float(W1.shape[0]))
    W2s = W2 / jnp.sqrt(float(W2.shape[0]))
    W3s = W3 / jnp.sqrt(float(W3.shape[0]))
    perm = jnp.asarray(np.concatenate([_SIGMA, F + _SIGMA]))
    W3p = W3s[:, perm]
    coef = _coef_pallas(vectors, W1s, W2s, W3p)
    coef2 = coef.reshape(4 * N_EDGES, F // 2)

    zeros = jnp.zeros((_ROWS_PER_TILE, F), jnp.float32)
    acc = _get_sc_kernel()(
        node_feats,
        senders.astype(jnp.int32),
        receivers.astype(jnp.int32),
        coef2,
        zeros,
    )

    acc = acc.reshape(4, _NP, F)[:, :N_NODES]
    out_s = acc[0]
    out_v = jnp.transpose(acc[1:], (1, 2, 0)).reshape(N_NODES, 3 * F)
    return jnp.concatenate([out_s, out_v], axis=1)


# speedup vs baseline: 1.4174x; 1.4174x over previous
import functools
from math import factorial

import jax
import jax.numpy as jnp
import numpy as np
from jax import lax
from jax.experimental import pallas as pl
from jax.experimental.pallas import tpu as pltpu
from jax.experimental.pallas import tpu_sc as plsc

N_NODES = 10000
N_EDGES = 320000
F = 128
N_BASIS = 8
HIDDEN = 64
AVG_NEIGH = 32.0


def _poly_env_coeffs(n0, n1):
    ncoef = n0 + n1 + 2
    A = np.zeros((ncoef, ncoef))
    b = np.zeros(ncoef)
    row = 0
    for k in range(n0 + 1):
        A[row, k] = float(factorial(k))
        b[row] = 1.0 if k == 0 else 0.0
        row += 1
    for k in range(n1 + 1):
        for j in range(k, ncoef):
            A[row, j] = float(factorial(j)) / float(factorial(j - k))
        b[row] = 0.0
        row += 1
    return np.linalg.solve(A, b)

_ENV_HORNER = tuple(float(c) for c in _poly_env_coeffs(5, 2)[::-1])

_z = np.random.RandomState(0).randn(1000000)
_ACT_NORM = float(1.0 / np.sqrt(np.mean((_z / (1.0 + np.exp(-_z))) ** 2)))

_SQRT3 = float(np.sqrt(3.0))
_SQRT2 = float(np.sqrt(2.0))
_PI = float(np.pi)
_INV_SQRT_NEIGH = float(1.0 / np.sqrt(AVG_NEIGH))


_BE = 1280


def _coef_body(v_ref, w1_ref, w2_ref, w3_ref, out_ref):
    v = v_ref[...]
    len2 = jnp.sum(v * v, axis=1, keepdims=True)
    length = jnp.sqrt(len2)
    safe = jnp.where(length == 0.0, 1.0, length)
    sh = (_SQRT3 / safe) * v

    n = lax.broadcasted_iota(jnp.int32, (_BE, N_BASIS), 1).astype(jnp.float32) + 1.0
    xnz = jnp.where(length == 0.0, 1.0, length)
    basis = _SQRT2 * jnp.where(
        length == 0.0, n * _PI, jnp.sin(n * (_PI * length)) / xnz
    )

    p = jnp.full_like(length, _ENV_HORNER[0])
    for cfl in _ENV_HORNER[1:]:
        p = p * length + cfl
    cutoff = jnp.where(length < 1.0, p, 0.0)

    radial = basis * cutoff

    def _swish(x):
        return _ACT_NORM * x / (1.0 + jnp.exp(-x))

    h = _swish(jnp.dot(radial, w1_ref[...], preferred_element_type=jnp.float32))
    h = _swish(jnp.dot(h, w2_ref[...], preferred_element_type=jnp.float32))
    mix = jnp.dot(h, w3_ref[...], preferred_element_type=jnp.float32)
    mix = mix * _INV_SQRT_NEIGH

    out_ref[0] = mix[:, :F]
    for k in range(3):
        out_ref[k + 1] = mix[:, F:] * sh[:, k : k + 1]


def _coef_pallas(vectors, W1s, W2s, W3s):
    grid = N_EDGES // _BE
    return pl.pallas_call(
        _coef_body,
        grid=(grid,),
        in_specs=[
            pl.BlockSpec((_BE, 3), lambda i: (i, 0)),
            pl.BlockSpec((N_BASIS, HIDDEN), lambda i: (0, 0)),
            pl.BlockSpec((HIDDEN, HIDDEN), lambda i: (0, 0)),
            pl.BlockSpec((HIDDEN, 2 * F), lambda i: (0, 0)),
        ],
        out_specs=pl.BlockSpec((4, _BE, F), lambda i: (0, i, 0)),
        out_shape=jax.ShapeDtypeStruct((4, N_EDGES, F), jnp.float32),
    )(vectors, W1s, W2s, W3s)


_NC = 2
_NS = 16
_C = 80
_EDGES_PER_TILE = N_EDGES // _NS
_NCHUNK = _EDGES_PER_TILE // _C
_NP = 10240
_ROWS_PER_TILE = _NP // _NS


def _sc_body(nf_hbm, send_hbm, recv_hbm, coef_hbm, zeros_hbm, out_hbm,
             s_v, r_v, nf_v, cf_v, msg_v, acc_sh, sem):
    c = lax.axis_index("c")
    s = lax.axis_index("s")
    row0 = s * _ROWS_PER_TILE

    for p in range(2):
        q = 2 * p + c

        pltpu.sync_copy(zeros_hbm, acc_sh.at[pl.ds(row0, _ROWS_PER_TILE)])
        plsc.subcore_barrier()

        def _edge_batch(j, carry):
            e0 = s * _EDGES_PER_TILE + j * _C
            pltpu.sync_copy(send_hbm.at[pl.ds(e0, _C)], s_v)
            pltpu.sync_copy(recv_hbm.at[pl.ds(e0, _C)], r_v)
            pltpu.sync_copy(coef_hbm.at[pl.ds(q * N_EDGES + e0, _C)], cf_v)
            pltpu.async_copy(nf_hbm.at[s_v], nf_v, sem).wait()

            def _mul_row(i, c2):
                for r8 in range(F // 16):
                    sl = pl.ds(r8 * 16, 16)
                    msg_v[i, sl] = nf_v[i, sl] * cf_v[i, sl]
                return c2

            lax.fori_loop(0, _C, _mul_row, 0)
            pltpu.sync_copy(msg_v, acc_sh.at[r_v], add=True)
            return carry

        lax.fori_loop(0, _NCHUNK, _edge_batch, 0)
        plsc.subcore_barrier()

        pltpu.sync_copy(
            acc_sh.at[pl.ds(row0, _ROWS_PER_TILE)],
            out_hbm.at[pl.ds(q * _NP + row0, _ROWS_PER_TILE)],
        )
        if p == 0:
            plsc.subcore_barrier()


@functools.cache
def _get_sc_kernel():
    return functools.partial(
        pl.kernel,
        mesh=plsc.VectorSubcoreMesh(core_axis_name="c", subcore_axis_name="s"),
        out_type=jax.ShapeDtypeStruct((4 * _NP, F), jnp.float32),
        scratch_types=[
            pltpu.VMEM((_C,), jnp.int32),
            pltpu.VMEM((_C,), jnp.int32),
            pltpu.VMEM((_C, F), jnp.float32),
            pltpu.VMEM((_C, F), jnp.float32),
            pltpu.VMEM((_C, F), jnp.float32),
            pltpu.VMEM_SHARED((_NP, F), jnp.float32),
            pltpu.SemaphoreType.DMA,
        ],
    )(_sc_body)


def kernel(vectors, node_feats, senders, receivers, W1, W2, W3):
    W1s = W1 / jnp.sqrt(float(W1.shape[0]))
    W2s = W2 / jnp.sqrt(float(W2.shape[0]))
    W3s = W3 / jnp.sqrt(float(W3.shape[0]))

    coef = _coef_pallas(vectors, W1s, W2s, W3s)
    coef2 = coef.reshape(4 * N_EDGES, F)

    zeros = jnp.zeros((_ROWS_PER_TILE, F), jnp.float32)
    acc = _get_sc_kernel()(
        node_feats,
        senders.astype(jnp.int32),
        receivers.astype(jnp.int32),
        coef2,
        zeros,
    )

    acc = acc.reshape(4, _NP, F)[:, :N_NODES]
    out_s = acc[0]
    out_v = jnp.transpose(acc[1:], (1, 2, 0)).reshape(N_NODES, 3 * F)
    return jnp.concatenate([out_s, out_v], axis=1)
